# Initial kernel scaffold; baseline (speedup 1.0000x reference)
import jax
import jax.numpy as jnp
from jax.experimental import pallas as pl


def kernel(x, edge_index, W1, a_src1, a_dst1, b1, W2, a_src2, a_dst2, b2):
    raise NotImplementedError("write your pallas kernel here")


# speedup vs baseline: 4.0282x; 4.0282x over previous
import functools

import jax
import jax.numpy as jnp
from jax import lax
from jax.experimental import pallas as pl
from jax.experimental.pallas import tpu as pltpu

N = 10000
E = 320000
IN = 128
HID = 64
HEADS = 2
OUT = 64


def _dense1_body(x_ref, w_ref, asrc_ref, adst_ref, h_ref, tbl_ref):
    h = jnp.dot(x_ref[...], w_ref[...], preferred_element_type=jnp.float32)
    h_ref[...] = h
    a_s = jnp.dot(h, asrc_ref[...], preferred_element_type=jnp.float32)
    a_d = jnp.dot(h, adst_ref[...], preferred_element_type=jnp.float32)
    tbl_ref[...] = jnp.concatenate([a_s, a_d], axis=1)


def _dense1(x, W1, a_src1, a_dst1):
    A_src = jnp.zeros((HEADS * HID, 8), jnp.float32)
    A_dst = jnp.zeros((HEADS * HID, 8), jnp.float32)
    for h in range(HEADS):
        A_src = A_src.at[h * HID:(h + 1) * HID, h].set(a_src1[h])
        A_dst = A_dst.at[h * HID:(h + 1) * HID, h].set(a_dst1[h])
    B = 1000
    h1, tbl = pl.pallas_call(
        _dense1_body,
        grid=(N // B,),
        in_specs=[
            pl.BlockSpec((B, IN), lambda i: (i, 0)),
            pl.BlockSpec((IN, HEADS * HID), lambda i: (0, 0)),
            pl.BlockSpec((HEADS * HID, 8), lambda i: (0, 0)),
            pl.BlockSpec((HEADS * HID, 8), lambda i: (0, 0)),
        ],
        out_specs=[
            pl.BlockSpec((B, HEADS * HID), lambda i: (i, 0)),
            pl.BlockSpec((B, 16), lambda i: (i, 0)),
        ],
        out_shape=[
            jax.ShapeDtypeStruct((N, HEADS * HID), jnp.float32),
            jax.ShapeDtypeStruct((N, 16), jnp.float32),
        ],
    )(x, W1, A_src, A_dst)
    return h1, tbl


def _edges_xla(h, tbl, src, dst, heads, ch):
    a_s = tbl[:, 0:heads][src]
    a_d = tbl[:, 8:8 + heads][dst]
    e = a_s + a_d
    e = jnp.where(e >= 0, e, 0.2 * e)
    ex = jnp.exp(e)
    hr = h[src].reshape(E, heads, ch)
    msg = hr * ex[:, :, None]
    num = jax.ops.segment_sum(msg.reshape(E, heads * ch), dst, num_segments=N)
    den = jax.ops.segment_sum(ex, dst, num_segments=N)
    return num, den


def _finish1_body(num_ref, den_ref, b1_ref, w2_ref, asrc2_ref, adst2_ref, h2_ref, tbl2_ref):
    den = den_ref[...]
    parts = []
    for h in range(HEADS):
        parts.append(num_ref[:, h * HID:(h + 1) * HID] / (den[:, h:h + 1] + 1e-16))
    h1 = jnp.concatenate(parts, axis=1) + b1_ref[...]
    h1 = jnp.maximum(h1, 0.0)
    h2 = jnp.dot(h1, w2_ref[...], preferred_element_type=jnp.float32)
    h2_ref[...] = h2
    a_s = jnp.dot(h2, asrc2_ref[...], preferred_element_type=jnp.float32)
    a_d = jnp.dot(h2, adst2_ref[...], preferred_element_type=jnp.float32)
    tbl2_ref[...] = jnp.concatenate([a_s, a_d], axis=1)


def _finish1(num, den, b1, W2, a_src2, a_dst2):
    B = 1000
    A_src = jnp.zeros((OUT, 8), jnp.float32).at[:, 0].set(a_src2[0])
    A_dst = jnp.zeros((OUT, 8), jnp.float32).at[:, 0].set(a_dst2[0])
    h2, tbl2 = pl.pallas_call(
        _finish1_body,
        grid=(N // B,),
        in_specs=[
            pl.BlockSpec((B, HEADS * HID), lambda i: (i, 0)),
            pl.BlockSpec((B, HEADS), lambda i: (i, 0)),
            pl.BlockSpec((1, HEADS * HID), lambda i: (0, 0)),
            pl.BlockSpec((HEADS * HID, OUT), lambda i: (0, 0)),
            pl.BlockSpec((OUT, 8), lambda i: (0, 0)),
            pl.BlockSpec((OUT, 8), lambda i: (0, 0)),
        ],
        out_specs=[
            pl.BlockSpec((B, OUT), lambda i: (i, 0)),
            pl.BlockSpec((B, 16), lambda i: (i, 0)),
        ],
        out_shape=[
            jax.ShapeDtypeStruct((N, OUT), jnp.float32),
            jax.ShapeDtypeStruct((N, 16), jnp.float32),
        ],
    )(num, den, b1.reshape(1, HEADS * HID), W2, A_src, A_dst)
    return h2, tbl2


def _finish2_body(num_ref, den_ref, b2_ref, out_ref):
    v = num_ref[...] / (den_ref[...] + 1e-16) + b2_ref[...]
    out_ref[...] = 1.0 / (1.0 + jnp.exp(-v))


def _finish2(num, den, b2):
    B = 1000
    return pl.pallas_call(
        _finish2_body,
        grid=(N // B,),
        in_specs=[
            pl.BlockSpec((B, OUT), lambda i: (i, 0)),
            pl.BlockSpec((B, 1), lambda i: (i, 0)),
            pl.BlockSpec((1, OUT), lambda i: (0, 0)),
        ],
        out_specs=pl.BlockSpec((B, OUT), lambda i: (i, 0)),
        out_shape=jax.ShapeDtypeStruct((N, OUT), jnp.float32),
    )(num, den, b2.reshape(1, OUT))


def kernel(x, edge_index, W1, a_src1, a_dst1, b1, W2, a_src2, a_dst2, b2):
    src = edge_index[0].astype(jnp.int32)
    dst = edge_index[1].astype(jnp.int32)
    h1, tbl1 = _dense1(x, W1, a_src1, a_dst1)
    num1, den1 = _edges_xla(h1, tbl1, src, dst, HEADS, HID)
    h2, tbl2 = _finish1(num1, den1, b1, W2, a_src2, a_dst2)
    num2, den2 = _edges_xla(h2, tbl2, src, dst, 1, OUT)
    return _finish2(num2, den2[:, 0:1], b2)


# speedup vs baseline: 31.8469x; 7.9060x over previous
import functools

import jax
import jax.numpy as jnp
from jax import lax
from jax.experimental import pallas as pl
from jax.experimental.pallas import tpu as pltpu
from jax.experimental.pallas import tpu_sc as plsc

N = 10000
E = 320000
IN = 128
HID = 64
HEADS = 2
OUT = 64

NC = 2
NS = 16
NW = NC * NS
K = 128
NCHUNK = E // K
GMAX = -(-NCHUNK // NW)
ROWS_PER_TILE = N // NS


def _dense1_body(x_ref, w_ref, asrc_ref, adst_ref, h_ref, tbl_ref):
    h = jnp.dot(x_ref[...], w_ref[...], preferred_element_type=jnp.float32)
    h_ref[...] = h
    a_s = jnp.dot(h, asrc_ref[...], preferred_element_type=jnp.float32)
    a_d = jnp.dot(h, adst_ref[...], preferred_element_type=jnp.float32)
    tbl_ref[...] = jnp.concatenate([a_s, a_d], axis=1)


def _dense1(x, W1, a_src1, a_dst1):
    A_src = jnp.zeros((HEADS * HID, 8), jnp.float32)
    A_dst = jnp.zeros((HEADS * HID, 8), jnp.float32)
    for h in range(HEADS):
        A_src = A_src.at[h * HID:(h + 1) * HID, h].set(a_src1[h])
        A_dst = A_dst.at[h * HID:(h + 1) * HID, h].set(a_dst1[h])
    B = 1000
    h1, tbl = pl.pallas_call(
        _dense1_body,
        grid=(N // B,),
        in_specs=[
            pl.BlockSpec((B, IN), lambda i: (i, 0)),
            pl.BlockSpec((IN, HEADS * HID), lambda i: (0, 0)),
            pl.BlockSpec((HEADS * HID, 8), lambda i: (0, 0)),
            pl.BlockSpec((HEADS * HID, 8), lambda i: (0, 0)),
        ],
        out_specs=[
            pl.BlockSpec((B, HEADS * HID), lambda i: (i, 0)),
            pl.BlockSpec((B, 16), lambda i: (i, 0)),
        ],
        out_shape=[
            jax.ShapeDtypeStruct((N, HEADS * HID), jnp.float32),
            jax.ShapeDtypeStruct((N, 16), jnp.float32),
        ],
    )(x, W1, A_src, A_dst)
    return h1, tbl


_SC_PARAMS = pltpu.CompilerParams(needs_layout_passes=False)


def _sc_pass1(tbl4, src, dst, H):
    mesh = plsc.VectorSubcoreMesh(core_axis_name="c", subcore_axis_name="s")

    @functools.partial(
        pl.kernel,
        mesh=mesh,
        compiler_params=_SC_PARAMS,
        out_type=[
            jax.ShapeDtypeStruct((NC, NS, H * N), jnp.float32),
            jax.ShapeDtypeStruct((E * H,), jnp.float32),
        ],
        scratch_types=[
            pltpu.VMEM((K,), jnp.int32),
            pltpu.VMEM((K,), jnp.int32),
            pltpu.VMEM((4 * N + 64,), jnp.float32),
            pltpu.VMEM((H * N,), jnp.float32),
            pltpu.VMEM((K * H,), jnp.float32),
        ],
    )
    def k(tbl_hbm, src_hbm, dst_hbm, den_hbm, ex_hbm,
          sidx, didx, tblv, denv, exbuf):
        cid = lax.axis_index("c")
        sid = lax.axis_index("s")
        wid = sid * NC + cid
        zero16 = jnp.zeros((16,), jnp.float32)

        pltpu.sync_copy(tbl_hbm, tblv)

        def _zden(r, carry):
            denv[pl.ds(r * 16, 16)] = zero16
            return carry
        lax.fori_loop(0, H * N // 16, _zden, 0)

        def _chunk(g, carry):
            cidx = g * NW + wid

            @pl.when(cidx < NCHUNK)
            def _():
                base = cidx * K
                pltpu.sync_copy(src_hbm.at[pl.ds(base, K)], sidx)
                pltpu.sync_copy(dst_hbm.at[pl.ds(base, K)], didx)

                def _jgrp(j, jcarry):
                    sv4 = sidx[pl.ds(j * 16, 16)] * 4
                    dv = didx[pl.ds(j * 16, 16)]
                    dv4 = dv * 4
                    for hh in range(H):
                        a_s = plsc.load_gather(tblv, [sv4 + hh])
                        a_d = plsc.load_gather(tblv, [dv4 + (2 + hh)])
                        e = a_s + a_d
                        e = jnp.where(e >= 0.0, e, 0.2 * e)
                        exv = jnp.exp(e)
                        plsc.addupdate_scatter(denv, [dv * H + hh], exv)
                        exbuf[pl.ds(hh * K + j * 16, 16)] = exv
                    return jcarry
                lax.fori_loop(0, K // 16, _jgrp, 0)
                pltpu.sync_copy(exbuf, ex_hbm.at[pl.ds(cidx * K * H, K * H)])
            return carry
        lax.fori_loop(0, GMAX, _chunk, 0)
        pltpu.sync_copy(denv, den_hbm.at[cid, sid])

    return k(tbl4, src, dst)


def _sc_pass2(h, exout, src, dst, H):
    mesh = plsc.VectorSubcoreMesh(core_axis_name="c", subcore_axis_name="s")

    @functools.partial(
        pl.kernel,
        mesh=mesh,
        compiler_params=_SC_PARAMS,
        out_type=jax.ShapeDtypeStruct((NC, N, 128), jnp.float32),
        scratch_types=[
            pltpu.VMEM((K,), jnp.int32),
            pltpu.VMEM((K,), jnp.int32),
            pltpu.VMEM((K * H,), jnp.float32),
            pltpu.VMEM((K, 128), jnp.float32),
            pltpu.VMEM_SHARED((N, 128), jnp.float32),
            pltpu.SemaphoreType.DMA,
        ],
    )
    def k(h_hbm, ex_hbm, src_hbm, dst_hbm, out_hbm,
          sidx, didx, exbuf, hbuf, acc, sem):
        cid = lax.axis_index("c")
        sid = lax.axis_index("s")
        wid = sid * NC + cid
        zero16 = jnp.zeros((16,), jnp.float32)

        def _zrow(r, carry):
            for c in range(8):
                hbuf[r, pl.ds(c * 16, 16)] = zero16
            return carry
        lax.fori_loop(0, K, _zrow, 0)
        for t in range(6):
            pltpu.sync_copy(
                hbuf.at[pl.ds(0, 104)],
                acc.at[pl.ds(sid * 624 + t * 104, 104)])

        @pl.when(sid == 0)
        def _ztail():
            pltpu.sync_copy(hbuf.at[pl.ds(0, 16)], acc.at[pl.ds(9984, 16)])
        plsc.subcore_barrier()

        def _chunk(g, carry):
            cidx = g * NW + wid

            @pl.when(cidx < NCHUNK)
            def _():
                base = cidx * K
                pltpu.sync_copy(src_hbm.at[pl.ds(base, K)], sidx)
                pltpu.sync_copy(dst_hbm.at[pl.ds(base, K)], didx)
                pltpu.sync_copy(ex_hbm.at[pl.ds(cidx * K * H, K * H)], exbuf)
                pltpu.async_copy(h_hbm.at[sidx], hbuf, sem).wait()

                def _jgrp(j, jcarry):
                    exg = [exbuf[pl.ds(hh * K + j * 16, 16)] for hh in range(H)]
                    for l in range(16):
                        row = j * 16 + l
                        for hh in range(H):
                            bex = jnp.full((16,), exg[hh][l], jnp.float32)
                            base_c = hh * (128 // H)
                            for c in range(128 // H // 16):
                                off = base_c + c * 16
                                hbuf[row, pl.ds(off, 16)] = (
                                    hbuf[row, pl.ds(off, 16)] * bex)
                    return jcarry
                lax.fori_loop(0, K // 16, _jgrp, 0)
                pltpu.sync_copy(hbuf, acc.at[didx], add=True)
            return carry
        lax.fori_loop(0, GMAX, _chunk, 0)

        plsc.subcore_barrier()
        pltpu.sync_copy(
            acc.at[pl.ds(sid * 624, 624)],
            out_hbm.at[cid, pl.ds(sid * 624, 624)])

        @pl.when(sid == 0)
        def _otail():
            pltpu.sync_copy(acc.at[pl.ds(9984, 16)],
                            out_hbm.at[cid, pl.ds(9984, 16)])

    return k(h, exout, src, dst)


def _sc_edge(h, tbl, src, dst, H):
    tbl4 = jnp.concatenate(
        [jnp.concatenate([tbl[:, 0:2], tbl[:, 8:10]], axis=1).reshape(-1),
         jnp.zeros((64,), jnp.float32)])
    den, exout = _sc_pass1(tbl4, src, dst, H)
    acc = _sc_pass2(h, exout, src, dst, H)
    return acc, den


def _finish1_body(n0_ref, n1_ref, den_ref, b1_ref, w2_ref, asrc2_ref,
                  adst2_ref, h2_ref, tbl2_ref):
    a = n0_ref[...] + n1_ref[...]
    den = jnp.sum(den_ref[...], axis=0)
    parts = []
    for h in range(HEADS):
        parts.append(a[:, h * HID:(h + 1) * HID]
                     / (den[:, h:h + 1] + 1e-16))
    h1 = jnp.concatenate(parts, axis=1) + b1_ref[...]
    h1 = jnp.maximum(h1, 0.0)
    h2 = jnp.dot(h1, w2_ref[...], preferred_element_type=jnp.float32)
    h2_ref[...] = h2
    a_s = jnp.dot(h2, asrc2_ref[...], preferred_element_type=jnp.float32)
    a_d = jnp.dot(h2, adst2_ref[...], preferred_element_type=jnp.float32)
    tbl2_ref[...] = jnp.concatenate([a_s, a_d], axis=1)


def _finish1(acc, den, b1, W2, a_src2, a_dst2):
    B = 1000
    acc2 = acc.reshape(NC * N, 128)
    den3 = den.reshape(NC * NS, N, HEADS)
    A_src = jnp.zeros((OUT, 8), jnp.float32).at[:, 0].set(a_src2[0])
    A_dst = jnp.zeros((OUT, 8), jnp.float32).at[:, 0].set(a_dst2[0])
    nb = N // B
    h2, tbl2 = pl.pallas_call(
        _finish1_body,
        grid=(nb,),
        in_specs=[
            pl.BlockSpec((B, 128), lambda i: (i, 0)),
            pl.BlockSpec((B, 128), lambda i, nb=nb: (nb + i, 0)),
            pl.BlockSpec((NC * NS, B, HEADS), lambda i: (0, i, 0)),
            pl.BlockSpec((1, HEADS * HID), lambda i: (0, 0)),
            pl.BlockSpec((HEADS * HID, OUT), lambda i: (0, 0)),
            pl.BlockSpec((OUT, 8), lambda i: (0, 0)),
            pl.BlockSpec((OUT, 8), lambda i: (0, 0)),
        ],
        out_specs=[
            pl.BlockSpec((B, OUT), lambda i: (i, 0)),
            pl.BlockSpec((B, 16), lambda i: (i, 0)),
        ],
        out_shape=[
            jax.ShapeDtypeStruct((N, OUT), jnp.float32),
            jax.ShapeDtypeStruct((N, 16), jnp.float32),
        ],
    )(acc2, acc2, den3, b1.reshape(1, HEADS * HID), W2, A_src, A_dst)
    return h2, tbl2


def _finish2_body(n0_ref, n1_ref, den_ref, b2_ref, out_ref):
    a = n0_ref[...] + n1_ref[...]
    den = jnp.sum(den_ref[...], axis=0)
    v = a[:, :OUT] / (den[:, 0:1] + 1e-16) + b2_ref[...]
    out_ref[...] = 1.0 / (1.0 + jnp.exp(-v))


def _finish2(acc, den, b2):
    B = 1000
    acc2 = acc.reshape(NC * N, 128)
    den3 = den.reshape(NC * NS, N, 1)
    nb = N // B
    return pl.pallas_call(
        _finish2_body,
        grid=(nb,),
        in_specs=[
            pl.BlockSpec((B, 128), lambda i: (i, 0)),
            pl.BlockSpec((B, 128), lambda i, nb=nb: (nb + i, 0)),
            pl.BlockSpec((NC * NS, B, 1), lambda i: (0, i, 0)),
            pl.BlockSpec((1, OUT), lambda i: (0, 0)),
        ],
        out_specs=pl.BlockSpec((B, OUT), lambda i: (i, 0)),
        out_shape=jax.ShapeDtypeStruct((N, OUT), jnp.float32),
    )(acc2, acc2, den3, b2.reshape(1, OUT))


def kernel(x, edge_index, W1, a_src1, a_dst1, b1, W2, a_src2, a_dst2, b2):
    src = edge_index[0].astype(jnp.int32)
    dst = edge_index[1].astype(jnp.int32)
    h1, tbl1 = _dense1(x, W1, a_src1, a_dst1)
    acc1, den1 = _sc_edge(h1, tbl1, src, dst, HEADS)
    h2, tbl2 = _finish1(acc1, den1, b1, W2, a_src2, a_dst2)
    h2p = jnp.concatenate([h2, jnp.zeros((N, 128 - OUT), jnp.float32)], axis=1)
    acc2, den2 = _sc_edge(h2p, tbl2, src, dst, 1)
    return _finish2(acc2, den2, b2)


# speedup vs baseline: 31.8554x; 1.0003x over previous
import functools

import jax
import jax.numpy as jnp
from jax import lax
from jax.experimental import pallas as pl
from jax.experimental.pallas import tpu as pltpu
from jax.experimental.pallas import tpu_sc as plsc

N = 10000
E = 320000
IN = 128
HID = 64
HEADS = 2
OUT = 64

NC = 2
NS = 16
NW = NC * NS
K = 128
NCHUNK = E // K
GMAX = -(-NCHUNK // NW)


def _dense1_body(x_ref, w_ref, asrc_ref, adst_ref, h_ref, tbl_ref):
    h = jnp.dot(x_ref[...], w_ref[...], preferred_element_type=jnp.float32)
    h_ref[...] = h
    a_s = jnp.dot(h, asrc_ref[...], preferred_element_type=jnp.float32)
    a_d = jnp.dot(h, adst_ref[...], preferred_element_type=jnp.float32)
    tbl_ref[...] = jnp.concatenate([a_s, a_d], axis=1)


def _dense1(x, W1, a_src1, a_dst1):
    A_src = jnp.zeros((HEADS * HID, 8), jnp.float32)
    A_dst = jnp.zeros((HEADS * HID, 8), jnp.float32)
    for h in range(HEADS):
        A_src = A_src.at[h * HID:(h + 1) * HID, h].set(a_src1[h])
        A_dst = A_dst.at[h * HID:(h + 1) * HID, h].set(a_dst1[h])
    B = 1000
    h1, tbl = pl.pallas_call(
        _dense1_body,
        grid=(N // B,),
        in_specs=[
            pl.BlockSpec((B, IN), lambda i: (i, 0)),
            pl.BlockSpec((IN, HEADS * HID), lambda i: (0, 0)),
            pl.BlockSpec((HEADS * HID, 8), lambda i: (0, 0)),
            pl.BlockSpec((HEADS * HID, 8), lambda i: (0, 0)),
        ],
        out_specs=[
            pl.BlockSpec((B, HEADS * HID), lambda i: (i, 0)),
            pl.BlockSpec((B, 16), lambda i: (i, 0)),
        ],
        out_shape=[
            jax.ShapeDtypeStruct((N, HEADS * HID), jnp.float32),
            jax.ShapeDtypeStruct((N, 16), jnp.float32),
        ],
    )(x, W1, A_src, A_dst)
    return h1, tbl


_SC_PARAMS = pltpu.CompilerParams(needs_layout_passes=False)


def _sc_pass1(tbl4, src, dst, H):
    mesh = plsc.VectorSubcoreMesh(core_axis_name="c", subcore_axis_name="s")

    @functools.partial(
        pl.kernel,
        mesh=mesh,
        compiler_params=_SC_PARAMS,
        out_type=[
            jax.ShapeDtypeStruct((NC, NS, H * N), jnp.float32),
            jax.ShapeDtypeStruct((E * H,), jnp.float32),
        ],
        scratch_types=[
            pltpu.VMEM((K,), jnp.int32),
            pltpu.VMEM((K,), jnp.int32),
            pltpu.VMEM((4 * N + 64,), jnp.float32),
            pltpu.VMEM((H * N,), jnp.float32),
            pltpu.VMEM((K * H,), jnp.float32),
        ],
    )
    def k(tbl_hbm, src_hbm, dst_hbm, den_hbm, ex_hbm,
          sidx, didx, tblv, denv, exbuf):
        cid = lax.axis_index("c")
        sid = lax.axis_index("s")
        wid = sid * NC + cid
        zero16 = jnp.zeros((16,), jnp.float32)

        pltpu.sync_copy(tbl_hbm, tblv)

        def _zden(r, carry):
            denv[pl.ds(r * 16, 16)] = zero16
            return carry
        lax.fori_loop(0, H * N // 16, _zden, 0)

        def _chunk(g, carry):
            cidx = g * NW + wid

            @pl.when(cidx < NCHUNK)
            def _():
                base = cidx * K
                pltpu.sync_copy(src_hbm.at[pl.ds(base, K)], sidx)
                pltpu.sync_copy(dst_hbm.at[pl.ds(base, K)], didx)

                def _jgrp(j, jcarry):
                    sv4 = sidx[pl.ds(j * 16, 16)] * 4
                    dv = didx[pl.ds(j * 16, 16)]
                    dv4 = dv * 4
                    for hh in range(H):
                        a_s = plsc.load_gather(tblv, [sv4 + hh])
                        a_d = plsc.load_gather(tblv, [dv4 + (2 + hh)])
                        e = a_s + a_d
                        e = jnp.where(e >= 0.0, e, 0.2 * e)
                        exv = jnp.exp(e)
                        plsc.addupdate_scatter(denv, [dv * H + hh], exv)
                        exbuf[pl.ds(hh * K + j * 16, 16)] = exv
                    return jcarry
                lax.fori_loop(0, K // 16, _jgrp, 0)
                pltpu.sync_copy(exbuf, ex_hbm.at[pl.ds(cidx * K * H, K * H)])
            return carry
        lax.fori_loop(0, GMAX, _chunk, 0)
        pltpu.sync_copy(denv, den_hbm.at[cid, sid])

    return k(tbl4, src, dst)


def _sc_pass2(h, exout, src, dst, H):
    mesh = plsc.VectorSubcoreMesh(core_axis_name="c", subcore_axis_name="s")

    @functools.partial(
        pl.kernel,
        mesh=mesh,
        compiler_params=_SC_PARAMS,
        out_type=jax.ShapeDtypeStruct((NC, N, 128), jnp.float32),
        scratch_types=[
            pltpu.VMEM((K,), jnp.int32),
            pltpu.VMEM((K,), jnp.int32),
            pltpu.VMEM((K * H,), jnp.float32),
            pltpu.VMEM((K, 128), jnp.float32),
            pltpu.VMEM_SHARED((N, 128), jnp.float32),
            pltpu.SemaphoreType.DMA,
        ],
    )
    def k(h_hbm, ex_hbm, src_hbm, dst_hbm, out_hbm,
          sidx, didx, exbuf, hbuf, acc, sem):
        cid = lax.axis_index("c")
        sid = lax.axis_index("s")
        wid = sid * NC + cid
        zero16 = jnp.zeros((16,), jnp.float32)

        def _zrow(r, carry):
            for c in range(8):
                hbuf[r, pl.ds(c * 16, 16)] = zero16
            return carry
        lax.fori_loop(0, K, _zrow, 0)
        for t in range(6):
            pltpu.sync_copy(
                hbuf.at[pl.ds(0, 104)],
                acc.at[pl.ds(sid * 624 + t * 104, 104)])

        @pl.when(sid == 0)
        def _ztail():
            pltpu.sync_copy(hbuf.at[pl.ds(0, 16)], acc.at[pl.ds(9984, 16)])
        plsc.subcore_barrier()

        def _chunk(g, carry):
            cidx = g * NW + wid

            @pl.when(cidx < NCHUNK)
            def _():
                base = cidx * K
                pltpu.sync_copy(src_hbm.at[pl.ds(base, K)], sidx)
                pltpu.sync_copy(dst_hbm.at[pl.ds(base, K)], didx)
                pltpu.sync_copy(ex_hbm.at[pl.ds(cidx * K * H, K * H)], exbuf)
                pltpu.async_copy(h_hbm.at[sidx], hbuf, sem).wait()

                def _jgrp(j, jcarry):
                    exg = [exbuf[pl.ds(hh * K + j * 16, 16)] for hh in range(H)]
                    for l in range(16):
                        row = j * 16 + l
                        for hh in range(H):
                            bex = jnp.full((16,), exg[hh][l], jnp.float32)
                            base_c = hh * (128 // H)
                            for c in range(128 // H // 16):
                                off = base_c + c * 16
                                hbuf[row, pl.ds(off, 16)] = (
                                    hbuf[row, pl.ds(off, 16)] * bex)
                    return jcarry
                lax.fori_loop(0, K // 16, _jgrp, 0)
                pltpu.sync_copy(hbuf, acc.at[didx], add=True)
            return carry
        lax.fori_loop(0, GMAX, _chunk, 0)

        plsc.subcore_barrier()
        pltpu.sync_copy(
            acc.at[pl.ds(sid * 624, 624)],
            out_hbm.at[cid, pl.ds(sid * 624, 624)])

        @pl.when(sid == 0)
        def _otail():
            pltpu.sync_copy(acc.at[pl.ds(9984, 16)],
                            out_hbm.at[cid, pl.ds(9984, 16)])

    return k(h, exout, src, dst)


def _sc_edge(h, tbl, src, dst, H):
    tbl4 = jnp.concatenate(
        [jnp.concatenate([tbl[:, 0:2], tbl[:, 8:10]], axis=1).reshape(-1),
         jnp.zeros((64,), jnp.float32)])
    den, exout = _sc_pass1(tbl4, src, dst, H)
    acc = _sc_pass2(h, exout, src, dst, H)
    return acc, den


def _finish1_body(n0_ref, n1_ref, den_ref, b1_ref, w2_ref, asrc2_ref,
                  adst2_ref, h2_ref, tbl2_ref):
    a = n0_ref[...] + n1_ref[...]
    den = jnp.sum(den_ref[...], axis=0)
    parts = []
    for h in range(HEADS):
        parts.append(a[:, h * HID:(h + 1) * HID]
                     / (den[:, h:h + 1] + 1e-16))
    h1 = jnp.concatenate(parts, axis=1) + b1_ref[...]
    h1 = jnp.maximum(h1, 0.0)
    h2 = jnp.dot(h1, w2_ref[...], preferred_element_type=jnp.float32)
    h2_ref[...] = h2
    a_s = jnp.dot(h2, asrc2_ref[...], preferred_element_type=jnp.float32)
    a_d = jnp.dot(h2, adst2_ref[...], preferred_element_type=jnp.float32)
    tbl2_ref[...] = jnp.concatenate([a_s, a_d], axis=1)


def _finish1(acc, den, b1, W2, a_src2, a_dst2):
    B = 1000
    acc2 = acc.reshape(NC * N, 128)
    den3 = den.reshape(NC * NS, N, HEADS)
    A_src = jnp.zeros((OUT, 8), jnp.float32).at[:, 0].set(a_src2[0])
    A_dst = jnp.zeros((OUT, 8), jnp.float32).at[:, 0].set(a_dst2[0])
    nb = N // B
    h2, tbl2 = pl.pallas_call(
        _finish1_body,
        grid=(nb,),
        in_specs=[
            pl.BlockSpec((B, 128), lambda i: (i, 0)),
            pl.BlockSpec((B, 128), lambda i, nb=nb: (nb + i, 0)),
            pl.BlockSpec((NC * NS, B, HEADS), lambda i: (0, i, 0)),
            pl.BlockSpec((1, HEADS * HID), lambda i: (0, 0)),
            pl.BlockSpec((HEADS * HID, OUT), lambda i: (0, 0)),
            pl.BlockSpec((OUT, 8), lambda i: (0, 0)),
            pl.BlockSpec((OUT, 8), lambda i: (0, 0)),
        ],
        out_specs=[
            pl.BlockSpec((B, OUT), lambda i: (i, 0)),
            pl.BlockSpec((B, 16), lambda i: (i, 0)),
        ],
        out_shape=[
            jax.ShapeDtypeStruct((N, OUT), jnp.float32),
            jax.ShapeDtypeStruct((N, 16), jnp.float32),
        ],
    )(acc2, acc2, den3, b1.reshape(1, HEADS * HID), W2, A_src, A_dst)
    return h2, tbl2


def _finish2_body(n0_ref, n1_ref, den_ref, b2_ref, out_ref):
    a = n0_ref[...] + n1_ref[...]
    den = jnp.sum(den_ref[...], axis=0)
    v = a[:, :OUT] / (den[:, 0:1] + 1e-16) + b2_ref[...]
    out_ref[...] = 1.0 / (1.0 + jnp.exp(-v))


def _finish2(acc, den, b2):
    B = 1000
    acc2 = acc.reshape(NC * N, 128)
    den3 = den.reshape(NC * NS, N, 1)
    nb = N // B
    return pl.pallas_call(
        _finish2_body,
        grid=(nb,),
        in_specs=[
            pl.BlockSpec((B, 128), lambda i: (i, 0)),
            pl.BlockSpec((B, 128), lambda i, nb=nb: (nb + i, 0)),
            pl.BlockSpec((NC * NS, B, 1), lambda i: (0, i, 0)),
            pl.BlockSpec((1, OUT), lambda i: (0, 0)),
        ],
        out_specs=pl.BlockSpec((B, OUT), lambda i: (i, 0)),
        out_shape=jax.ShapeDtypeStruct((N, OUT), jnp.float32),
    )(acc2, acc2, den3, b2.reshape(1, OUT))


def kernel(x, edge_index, W1, a_src1, a_dst1, b1, W2, a_src2, a_dst2, b2):
    src = edge_index[0].astype(jnp.int32)
    dst = edge_index[1].astype(jnp.int32)
    h1, tbl1 = _dense1(x, W1, a_src1, a_dst1)
    acc1, den1 = _sc_edge(h1, tbl1, src, dst, HEADS)
    h2, tbl2 = _finish1(acc1, den1, b1, W2, a_src2, a_dst2)
    h2p = jnp.concatenate([h2, jnp.zeros((N, 128 - OUT), jnp.float32)], axis=1)
    acc2, den2 = _sc_edge(h2p, tbl2, src, dst, 1)
    return _finish2(acc2, den2, b2)
